# baseline (speedup 1.0000x reference)
import functools
import math

import jax
import jax.numpy as jnp
from jax.experimental import pallas as pl


def _loss_body(t_ref, x_ref, o_ref, *, bc, ncols, nblk):
    j = pl.program_id(0)
    rows = x_ref.shape[0]
    cols = jax.lax.broadcasted_iota(jnp.int32, (rows, bc), 1)
    t_loc = t_ref[...] - j * bc

    def accum(s):
        @pl.when(j == 0)
        def _():
            o_ref[...] = s

        @pl.when(j > 0)
        def _():
            o_ref[...] += s

    @pl.when(j < nblk - 1)
    def _main():
        logs = jnp.log2(x_ref[...])
        accum(jnp.sum(jnp.where(cols == t_loc, 0.0, logs),
                      axis=1, keepdims=True))

    @pl.when(j == nblk - 1)
    def _last():
        nvalid = ncols - (nblk - 1) * bc
        logs = jnp.log2(x_ref[...])
        accum(jnp.sum(jnp.where((cols == t_loc) | (cols >= nvalid), 0.0, logs),
                      axis=1, keepdims=True))
        o_ref[...] = o_ref[...] * (-math.log(2.0) / ncols)


def kernel(pred, target):
    B, C = pred.shape
    BC = 6144
    nblk = pl.cdiv(C, BC)
    t2 = target.astype(jnp.int32).reshape(B, 1)
    out = pl.pallas_call(
        functools.partial(_loss_body, bc=BC, ncols=C, nblk=nblk),
        grid=(nblk,),
        in_specs=[
            pl.BlockSpec((B, 1), lambda j: (0, 0)),
            pl.BlockSpec((B, BC), lambda j: (0, j)),
        ],
        out_specs=pl.BlockSpec((B, 1), lambda j: (0, 0)),
        out_shape=jax.ShapeDtypeStruct((B, 1), jnp.float32),
    )(t2, pred)
    return out[:, 0]


# speedup vs baseline: 1.0025x; 1.0025x over previous
import functools
import math

import jax
import jax.numpy as jnp
from jax.experimental import pallas as pl


def _loss_body(t_ref, x_ref, o_ref, *, bc, ncols, nblk):
    j = pl.program_id(0)
    rows = x_ref.shape[0]
    cols = jax.lax.broadcasted_iota(jnp.int32, (rows, bc), 1)
    t_loc = t_ref[...] - j * bc

    def accum(s):
        @pl.when(j == 0)
        def _():
            o_ref[...] = s

        @pl.when(j > 0)
        def _():
            o_ref[...] += s

    @pl.when(j < nblk - 1)
    def _main():
        logs = jnp.log2(x_ref[...])
        accum(jnp.sum(jnp.where(cols == t_loc, 0.0, logs),
                      axis=1, keepdims=True))

    @pl.when(j == nblk - 1)
    def _last():
        nvalid = ncols - (nblk - 1) * bc
        logs = jnp.log2(x_ref[...])
        accum(jnp.sum(jnp.where((cols == t_loc) | (cols >= nvalid), 0.0, logs),
                      axis=1, keepdims=True))
        o_ref[...] = o_ref[...] * (-math.log(2.0) / ncols)


def kernel(pred, target):
    B, C = pred.shape
    BC = 4096
    nblk = pl.cdiv(C, BC)
    t2 = target.astype(jnp.int32).reshape(B, 1)
    out = pl.pallas_call(
        functools.partial(_loss_body, bc=BC, ncols=C, nblk=nblk),
        grid=(nblk,),
        in_specs=[
            pl.BlockSpec((B, 1), lambda j: (0, 0)),
            pl.BlockSpec((B, BC), lambda j: (0, j)),
        ],
        out_specs=pl.BlockSpec((B, 1), lambda j: (0, 0)),
        out_shape=jax.ShapeDtypeStruct((B, 1), jnp.float32),
    )(t2, pred)
    return out[:, 0]
